# Initial kernel scaffold; baseline (speedup 1.0000x reference)
import jax
import jax.numpy as jnp
from jax.experimental import pallas as pl


def kernel(x_nchw, c1_w, c1_b, c2_w, c2_b, c3_w, c3_b, res0_w1, res0_w2, res1_w1, res1_w2):
    raise NotImplementedError("write your pallas kernel here")


# speedup vs baseline: 1.6366x; 1.6366x over previous
import functools

import jax
import jax.numpy as jnp
from jax.experimental import pallas as pl
from jax.experimental.pallas import tpu as pltpu

_BF = jnp.bfloat16

_BB1 = 2
_BB2 = 4
_BB3 = 8


def _s2d(x):
    B, H, W, C = x.shape
    xp = jnp.pad(x, ((0, 0), (1, 1), (1, 1), (0, 0)))
    Hi, Wi = (H + 2) // 2, (W + 2) // 2
    xp = xp.reshape(B, Hi, 2, Wi, 2, C)
    xp = jnp.transpose(xp, (0, 1, 3, 2, 4, 5))
    return xp.reshape(B, Hi, Wi, 4 * C)


def _conv1_body(x_ref, w_ref, b_ref, o_ref, *, bb):
    x = x_ref[...]
    cols = []
    for a in range(2):
        for b in range(2):
            sl = x[:, a:a + 64, b:b + 64, :]
            cols.append(sl.reshape(bb * 64 * 64, 12))
    patches = jnp.concatenate(cols, axis=-1)
    acc = jnp.dot(patches, w_ref[...], preferred_element_type=jnp.float32)
    acc = jnp.maximum(acc + b_ref[...], 0.0)
    o_ref[...] = acc.reshape(o_ref.shape).astype(o_ref.dtype)


def _conv2_body(x_ref, w_ref, b_ref, o_ref, *, bb, hw, relu):
    x = x_ref[...]
    acc = b_ref[...]
    for a in range(2):
        for b in range(2):
            sl = x[:, a:a + hw, b:b + hw, :].reshape(bb * hw * hw, 512)
            acc = acc + jnp.dot(sl, w_ref[2 * a + b],
                                preferred_element_type=jnp.float32)
    if relu:
        acc = jnp.maximum(acc, 0.0)
    o_ref[...] = acc.reshape(o_ref.shape).astype(o_ref.dtype)


def _tail_body(x_ref, w3_ref, b3_ref, r0w1_ref, r0w2_ref, r1w1_ref, r1w2_ref,
               o_ref, pad_ref, *, bb):
    x = x_ref[...]
    acc = b3_ref[...]
    for a in range(2):
        for b in range(2):
            sl = x[:, a:a + 16, b:b + 16, :].reshape(bb * 256, 512)
            acc = acc + jnp.dot(sl, w3_ref[2 * a + b],
                                preferred_element_type=jnp.float32)
    h = acc

    pad_ref[...] = jnp.zeros(pad_ref.shape, _BF)
    for w1_ref, w2_ref, relu_out in ((r0w1_ref, r0w2_ref, False),
                                     (r1w1_ref, r1w2_ref, True)):
        hr = jnp.maximum(h, 0.0).astype(_BF)
        pad_ref[:, 1:17, 1:17, :] = hr.reshape(bb, 16, 16, 128)
        xp = pad_ref[...]
        t = None
        for kh in range(3):
            for kw in range(3):
                sl = xp[:, kh:kh + 16, kw:kw + 16, :].reshape(bb * 256, 128)
                d = jnp.dot(sl, w1_ref[3 * kh + kw],
                            preferred_element_type=jnp.float32)
                t = d if t is None else t + d
        t = jnp.maximum(t, 0.0).astype(_BF)
        y = jnp.dot(t, w2_ref[...], preferred_element_type=jnp.float32)
        h = h + y
        if relu_out:
            h = jnp.maximum(h, 0.0)
    o_ref[...] = h.reshape(o_ref.shape).astype(o_ref.dtype)


def _pcall(body, grid, in_specs, out_shape, out_spec, scratch_shapes=()):
    return pl.pallas_call(
        body,
        grid=grid,
        in_specs=in_specs,
        out_shape=out_shape,
        out_specs=out_spec,
        scratch_shapes=list(scratch_shapes),
        compiler_params=pltpu.CompilerParams(
            dimension_semantics=("parallel",)),
    )


def kernel(x_nchw, c1_w, c1_b, c2_w, c2_b, c3_w, c3_b,
           res0_w1, res0_w2, res1_w1, res1_w2):
    B = x_nchw.shape[0]
    h = jnp.transpose(x_nchw, (0, 2, 3, 1))
    xs1 = _s2d(h).astype(_BF)

    w1 = c1_w.astype(_BF)
    w2 = c2_w.astype(_BF).reshape(4, 512, 128)
    w3 = c3_w.astype(_BF).reshape(4, 512, 128)
    r0w1 = res0_w1.astype(_BF).reshape(9, 128, 128)
    r0w2 = res0_w2.astype(_BF)
    r1w1 = res1_w1.astype(_BF).reshape(9, 128, 128)
    r1w2 = res1_w2.astype(_BF)

    full = lambda shp: pl.BlockSpec(shp, lambda i: (0,) * len(shp))

    h1 = _pcall(
        functools.partial(_conv1_body, bb=_BB1),
        grid=(B // _BB1,),
        in_specs=[
            pl.BlockSpec((_BB1, 65, 65, 12), lambda i: (i, 0, 0, 0)),
            full((48, 128)), full((1, 128)),
        ],
        out_shape=jax.ShapeDtypeStruct((B, 64, 64, 128), _BF),
        out_spec=pl.BlockSpec((_BB1, 64, 64, 128), lambda i: (i, 0, 0, 0)),
    )(xs1, w1, c1_b)

    xs2 = _s2d(h1)
    h2 = _pcall(
        functools.partial(_conv2_body, bb=_BB2, hw=32, relu=True),
        grid=(B // _BB2,),
        in_specs=[
            pl.BlockSpec((_BB2, 33, 33, 512), lambda i: (i, 0, 0, 0)),
            full((4, 512, 128)), full((1, 128)),
        ],
        out_shape=jax.ShapeDtypeStruct((B, 32, 32, 128), _BF),
        out_spec=pl.BlockSpec((_BB2, 32, 32, 128), lambda i: (i, 0, 0, 0)),
    )(xs2, w2, c2_b)

    xs3 = _s2d(h2)
    out = _pcall(
        functools.partial(_tail_body, bb=_BB3),
        grid=(B // _BB3,),
        in_specs=[
            pl.BlockSpec((_BB3, 17, 17, 512), lambda i: (i, 0, 0, 0)),
            full((4, 512, 128)), full((1, 128)),
            full((9, 128, 128)), full((128, 128)),
            full((9, 128, 128)), full((128, 128)),
        ],
        out_shape=jax.ShapeDtypeStruct((B, 16, 16, 128), jnp.float32),
        out_spec=pl.BlockSpec((_BB3, 16, 16, 128), lambda i: (i, 0, 0, 0)),
        scratch_shapes=[pltpu.VMEM((_BB3, 18, 18, 128), _BF)],
    )(xs3, w3, c3_b, r0w1, r0w2, r1w1, r1w2)

    return jnp.transpose(out, (0, 3, 1, 2))


# speedup vs baseline: 2.7859x; 1.7022x over previous
import functools

import jax
import jax.numpy as jnp
from jax.experimental import pallas as pl
from jax.experimental.pallas import tpu as pltpu

_BF = jnp.bfloat16
_BB = 4


def _s2d(x):
    B, H, W, C = x.shape
    xp = jnp.pad(x, ((0, 0), (1, 1), (1, 1), (0, 0)))
    Hi, Wi = (H + 2) // 2, (W + 2) // 2
    xp = xp.reshape(B, Hi, 2, Wi, 2, C)
    xp = jnp.transpose(xp, (0, 1, 3, 2, 4, 5))
    return xp.reshape(B, Hi, Wi, 4 * C)


def _repack(h, dst_ref, bb, hw):
    hv = h.reshape(bb, hw, 2, hw, 256)
    for dh in (0, 1):
        for dw in (0, 1):
            c0 = 128 * (2 * dh + dw)
            q = 1 - dw
            csl = hv[:, :, 1 - dh, :, q * 128:(q + 1) * 128]
            u0, v0 = 1 - dh, 1 - dw
            ur = (hw, hw + 1) if dh else (0, 1)
            vr = (hw, hw + 1) if dw else (0, 1)
            dst_ref[:, ur[0]:ur[1], :, c0:c0 + 128] = jnp.zeros(
                (bb, 1, hw + 1, 128), _BF)
            dst_ref[:, :, vr[0]:vr[1], c0:c0 + 128] = jnp.zeros(
                (bb, hw + 1, 1, 128), _BF)
            dst_ref[:, u0:u0 + hw, v0:v0 + hw, c0:c0 + 128] = csl


def _sconv(x, w_ref, bias, bb, hw):
    acc = bias
    for a in range(2):
        for b in range(2):
            sl = x[:, a:a + hw, b:b + hw, :].reshape(bb * hw * hw, 512)
            acc = acc + jnp.dot(sl, w_ref[2 * a + b],
                                preferred_element_type=jnp.float32)
    return acc


def _mega_body(p1_ref, w1_ref, b1_ref, w2_ref, b2_ref, w3_ref, b3_ref,
               r0w1_ref, r0w2_ref, r1w1_ref, r1w2_ref, o_ref,
               h1_ref, xs2_ref, xs3_ref, pad_ref, *, bb):
    acc = jnp.dot(p1_ref[...].reshape(bb * 4096, 48), w1_ref[...],
                  preferred_element_type=jnp.float32)
    acc = jnp.maximum(acc + b1_ref[...], 0.0)
    h1_ref[...] = acc.reshape(bb, 64, 64, 128).astype(_BF)

    _repack(h1_ref[...], xs2_ref, bb, 32)
    acc = _sconv(xs2_ref[...], w2_ref, b2_ref[...], bb, 32)
    h2 = jnp.maximum(acc, 0.0).astype(_BF).reshape(bb, 32, 32, 128)

    _repack(h2, xs3_ref, bb, 16)
    h = _sconv(xs3_ref[...], w3_ref, b3_ref[...], bb, 16)

    for w1_ref, w2_ref, relu_out in ((r0w1_ref, r0w2_ref, False),
                                     (r1w1_ref, r1w2_ref, True)):
        hr = jnp.maximum(h, 0.0).astype(_BF)
        pad_ref[...] = jnp.zeros(pad_ref.shape, _BF)
        pad_ref[:, 1:17, 1:17, :] = hr.reshape(bb, 16, 16, 128)
        xp = pad_ref[...]
        t = None
        for kh in range(3):
            for kw in range(3):
                sl = xp[:, kh:kh + 16, kw:kw + 16, :].reshape(bb * 256, 128)
                d = jnp.dot(sl, w1_ref[3 * kh + kw],
                            preferred_element_type=jnp.float32)
                t = d if t is None else t + d
        t = jnp.maximum(t, 0.0).astype(_BF)
        h = h + jnp.dot(t, w2_ref[...], preferred_element_type=jnp.float32)
        if relu_out:
            h = jnp.maximum(h, 0.0)

    o_ref[...] = jnp.transpose(h.reshape(bb, 256, 128), (0, 2, 1))


def kernel(x_nchw, c1_w, c1_b, c2_w, c2_b, c3_w, c3_b,
           res0_w1, res0_w2, res1_w1, res1_w2):
    B = x_nchw.shape[0]
    h = jnp.transpose(x_nchw, (0, 2, 3, 1))
    xs1 = _s2d(h).astype(_BF)
    cols = [xs1[:, a:a + 64, b:b + 64, :] for a in range(2) for b in range(2)]
    p1 = jnp.concatenate(cols, axis=-1).reshape(B, 4096, 48)

    w1 = c1_w.astype(_BF)
    w2 = c2_w.astype(_BF).reshape(4, 512, 128)
    w3 = c3_w.astype(_BF).reshape(4, 512, 128)
    r0w1 = res0_w1.astype(_BF).reshape(9, 128, 128)
    r0w2 = res0_w2.astype(_BF)
    r1w1 = res1_w1.astype(_BF).reshape(9, 128, 128)
    r1w2 = res1_w2.astype(_BF)

    full = lambda shp: pl.BlockSpec(shp, lambda i: (0,) * len(shp))

    out = pl.pallas_call(
        functools.partial(_mega_body, bb=_BB),
        grid=(B // _BB,),
        in_specs=[
            pl.BlockSpec((_BB, 4096, 48), lambda i: (i, 0, 0)),
            full((48, 128)), full((1, 128)),
            full((4, 512, 128)), full((1, 128)),
            full((4, 512, 128)), full((1, 128)),
            full((9, 128, 128)), full((128, 128)),
            full((9, 128, 128)), full((128, 128)),
        ],
        out_shape=jax.ShapeDtypeStruct((B, 128, 256), jnp.float32),
        out_specs=pl.BlockSpec((_BB, 128, 256), lambda i: (i, 0, 0)),
        scratch_shapes=[
            pltpu.VMEM((_BB, 64, 64, 128), _BF),
            pltpu.VMEM((_BB, 33, 33, 512), _BF),
            pltpu.VMEM((_BB, 17, 17, 512), _BF),
            pltpu.VMEM((_BB, 18, 18, 128), _BF),
        ],
        compiler_params=pltpu.CompilerParams(
            dimension_semantics=("parallel",)),
    )(p1, w1, c1_b, w2, c2_b, w3, c3_b, r0w1, r0w2, r1w1, r1w2)

    return out.reshape(B, 128, 16, 16)
